# baseline (device time: 42795 ns/iter reference)
import jax
import jax.numpy as jnp
from jax import lax
from jax.experimental import pallas as pl
from jax.experimental.pallas import tpu as pltpu

N_DEV = 4
N_TOK = 1024
D_IN = 512
D_OUT = 1024
N_EXP = 16
E_LOC = N_EXP // N_DEV
N_HOPS = N_DEV - 1
CHUNK = N_TOK // N_DEV
COLS = D_OUT // 2


def kernel(x, router_W, route_idx, expert_W):
    def body(x_ref, rw_ref, idx_ref, ew_ref, out_ref,
             w_ref, rs_send, rs_recv, ag_buf,
             rs_ssem, rs_rsem, ag_ssem, ag_rsem):
        my = lax.axis_index("i")
        left = lax.rem(my + N_DEV - 1, N_DEV)
        right = lax.rem(my + 1, N_DEV)
        peer = (right, left)

        barrier = pltpu.get_barrier_semaphore()
        for nbr in (left, right):
            pl.semaphore_signal(
                barrier, inc=1, device_id=(nbr,),
                device_id_type=pl.DeviceIdType.MESH,
            )
        pl.semaphore_wait(barrier, 2)

        xv = x_ref[:, :]
        scores = jnp.dot(xv, rw_ref[:, :], preferred_element_type=jnp.float32)
        p = jnp.exp(scores - jnp.max(scores, axis=-1, keepdims=True))
        p = p / jnp.sum(p, axis=-1, keepdims=True)
        idx = idx_ref[:, :]
        eids = lax.broadcasted_iota(jnp.int32, (N_TOK, N_EXP), 1)
        g0 = jnp.sum(jnp.where(eids == idx[:, 0:1], p, 0.0), axis=-1, keepdims=True)
        g1 = jnp.sum(jnp.where(eids == idx[:, 1:2], p, 0.0), axis=-1, keepdims=True)
        inv = 1.0 / (g0 + g1)
        for k in range(E_LOC):
            e_k = my * E_LOC + k
            w_ref[:, k:k + 1] = (
                jnp.where(idx[:, 0:1] == e_k, g0 * inv, 0.0)
                + jnp.where(idx[:, 1:2] == e_k, g1 * inv, 0.0)
            )
        ew = jnp.reshape(ew_ref[:, :, :], (E_LOC * D_IN, D_OUT)).astype(jnp.bfloat16)

        def gated_chunk(c):
            xc = x_ref[pl.ds(c * CHUNK, CHUNK), :]
            wc = w_ref[pl.ds(c * CHUNK, CHUNK), :]
            return jnp.concatenate(
                [(wc[:, k:k + 1] * xc).astype(jnp.bfloat16) for k in range(E_LOC)],
                axis=1,
            )

        def cidx(r, s):
            return lax.rem(my + (N_DEV - s if r == 0 else s), N_DEV)

        def rs_hop(r, s):
            return pltpu.make_async_remote_copy(
                src_ref=rs_send.at[r, s], dst_ref=rs_recv.at[r, s],
                send_sem=rs_ssem.at[r, s], recv_sem=rs_rsem.at[r, s],
                device_id=(peer[r],), device_id_type=pl.DeviceIdType.MESH,
            )

        def ag_hop(r, h):
            return pltpu.make_async_remote_copy(
                src_ref=ag_buf.at[r, h], dst_ref=ag_buf.at[r, h + 1],
                send_sem=ag_ssem.at[r, h], recv_sem=ag_rsem.at[r, h],
                device_id=(peer[r],), device_id_type=pl.DeviceIdType.MESH,
            )

        p0 = jnp.dot(gated_chunk(my), ew, preferred_element_type=jnp.float32)
        rs = {0: [], 1: []}
        ag = {0: [], 1: []}
        for r in (0, 1):
            rs_send[r, 0, :, :] = p0[:, r * COLS:(r + 1) * COLS].astype(jnp.bfloat16)
            d = rs_hop(r, 0)
            d.start()
            rs[r].append(d)
        for s in range(1, N_DEV):
            acc = {r: jnp.dot(gated_chunk(cidx(r, s)),
                              ew[:, r * COLS:(r + 1) * COLS],
                              preferred_element_type=jnp.float32)
                   for r in (0, 1)}
            for r in (0, 1):
                rs[r][s - 1].wait_recv()
                merged = rs_recv[r, s - 1].astype(jnp.float32) + acc[r]
                if s < N_DEV - 1:
                    rs_send[r, s, :, :] = merged.astype(jnp.bfloat16)
                    d = rs_hop(r, s)
                    d.start()
                    rs[r].append(d)
                else:
                    ag_buf[r, 0, :, :] = merged.astype(jnp.bfloat16)
                    d = ag_hop(r, 0)
                    d.start()
                    ag[r].append(d)
                    out_ref[pl.ds(cidx(r, 3) * CHUNK, CHUNK),
                            r * COLS:(r + 1) * COLS] = merged

        for h in range(N_HOPS):
            for r in (0, 1):
                ag[r][h].wait_recv()
                if h < N_HOPS - 1:
                    d = ag_hop(r, h + 1)
                    d.start()
                    ag[r].append(d)
                out_ref[pl.ds(cidx(r, h) * CHUNK, CHUNK),
                        r * COLS:(r + 1) * COLS] = (
                            ag_buf[r, h + 1].astype(jnp.float32))

        for r in (0, 1):
            for d in rs[r] + ag[r]:
                d.wait_send()

    return pl.pallas_call(
        body,
        out_shape=jax.ShapeDtypeStruct((N_TOK, D_OUT), jnp.float32),
        in_specs=[pl.BlockSpec(memory_space=pltpu.VMEM)] * 4,
        out_specs=pl.BlockSpec(memory_space=pltpu.VMEM),
        scratch_shapes=[
            pltpu.VMEM((N_TOK, E_LOC), jnp.float32),
            pltpu.VMEM((2, N_HOPS, CHUNK, COLS), jnp.bfloat16),
            pltpu.VMEM((2, N_HOPS, CHUNK, COLS), jnp.bfloat16),
            pltpu.VMEM((2, N_DEV, CHUNK, COLS), jnp.bfloat16),
            pltpu.SemaphoreType.DMA((2, N_HOPS)),
            pltpu.SemaphoreType.DMA((2, N_HOPS)),
            pltpu.SemaphoreType.DMA((2, N_HOPS)),
            pltpu.SemaphoreType.DMA((2, N_HOPS)),
        ],
        compiler_params=pltpu.CompilerParams(collective_id=0),
    )(x, router_W, route_idx, expert_W)


# device time: 41001 ns/iter; 1.0438x vs baseline; 1.0438x over previous
import jax
import jax.numpy as jnp
from jax import lax
from jax.experimental import pallas as pl
from jax.experimental.pallas import tpu as pltpu

N_DEV = 4
N_TOK = 1024
D_IN = 512
D_OUT = 1024
N_EXP = 16
E_LOC = N_EXP // N_DEV
N_HOPS = N_DEV - 1
CHUNK = N_TOK // N_DEV
COLS = D_OUT // 2


def kernel(x, router_W, route_idx, expert_W):
    def body(x_ref, rw_ref, idx_ref, ew_ref, out_ref,
             w_ref, rs_send, rs_recv, ag_buf,
             rs_ssem, rs_rsem, ag_ssem, ag_rsem):
        my = lax.axis_index("i")
        left = lax.rem(my + N_DEV - 1, N_DEV)
        right = lax.rem(my + 1, N_DEV)
        peer = (right, left)

        barrier = pltpu.get_barrier_semaphore()
        for nbr in (left, right):
            pl.semaphore_signal(
                barrier, inc=1, device_id=(nbr,),
                device_id_type=pl.DeviceIdType.MESH,
            )
        pl.semaphore_wait(barrier, 2)

        def fill_gates(off, n):
            xc = x_ref[pl.ds(off, n), :]
            scores = jnp.dot(xc, rw_ref[:, :], preferred_element_type=jnp.float32)
            p = jnp.exp(scores - jnp.max(scores, axis=-1, keepdims=True))
            p = p / jnp.sum(p, axis=-1, keepdims=True)
            idx = idx_ref[pl.ds(off, n), :]
            eids = lax.broadcasted_iota(jnp.int32, (n, N_EXP), 1)
            g0 = jnp.sum(jnp.where(eids == idx[:, 0:1], p, 0.0),
                         axis=-1, keepdims=True)
            g1 = jnp.sum(jnp.where(eids == idx[:, 1:2], p, 0.0),
                         axis=-1, keepdims=True)
            inv = 1.0 / (g0 + g1)
            for k in range(E_LOC):
                e_k = my * E_LOC + k
                w_ref[pl.ds(off, n), k:k + 1] = (
                    jnp.where(idx[:, 0:1] == e_k, g0 * inv, 0.0)
                    + jnp.where(idx[:, 1:2] == e_k, g1 * inv, 0.0)
                )

        def gated_chunk(c):
            xc = x_ref[pl.ds(c * CHUNK, CHUNK), :]
            wc = w_ref[pl.ds(c * CHUNK, CHUNK), :]
            return jnp.concatenate(
                [(wc[:, k:k + 1] * xc).astype(jnp.bfloat16) for k in range(E_LOC)],
                axis=1,
            )

        def cidx(r, s):
            return lax.rem(my + (N_DEV - s if r == 0 else s), N_DEV)

        def rs_hop(r, s):
            return pltpu.make_async_remote_copy(
                src_ref=rs_send.at[r, s], dst_ref=rs_recv.at[r, s],
                send_sem=rs_ssem.at[r, s], recv_sem=rs_rsem.at[r, s],
                device_id=(peer[r],), device_id_type=pl.DeviceIdType.MESH,
            )

        def ag_hop(r, h):
            return pltpu.make_async_remote_copy(
                src_ref=ag_buf.at[r, h], dst_ref=ag_buf.at[r, h + 1],
                send_sem=ag_ssem.at[r, h], recv_sem=ag_rsem.at[r, h],
                device_id=(peer[r],), device_id_type=pl.DeviceIdType.MESH,
            )

        fill_gates(my * CHUNK, CHUNK)
        ew_full = jnp.reshape(ew_ref[:, :, :], (E_LOC * D_IN, D_OUT))
        ew = [ew_full[:, r * COLS:(r + 1) * COLS].astype(jnp.bfloat16)
              for r in (0, 1)]
        gm = gated_chunk(my)
        rs = {0: [], 1: []}
        ag = {0: [], 1: []}
        for r in (0, 1):
            rs_send[r, 0, :, :] = jnp.dot(
                gm, ew[r], preferred_element_type=jnp.float32
            ).astype(jnp.bfloat16)
            d = rs_hop(r, 0)
            d.start()
            rs[r].append(d)

        fill_gates(lax.rem(my + 1, N_DEV) * CHUNK, CHUNK)
        fill_gates(lax.rem(my + 2, N_DEV) * CHUNK, CHUNK)
        fill_gates(lax.rem(my + 3, N_DEV) * CHUNK, CHUNK)

        for s in range(1, N_DEV):
            acc = {r: jnp.dot(gated_chunk(cidx(r, s)), ew[r],
                              preferred_element_type=jnp.float32)
                   for r in (0, 1)}
            for r in (0, 1):
                rs[r][s - 1].wait_recv()
                merged = rs_recv[r, s - 1].astype(jnp.float32) + acc[r]
                if s < N_DEV - 1:
                    rs_send[r, s, :, :] = merged.astype(jnp.bfloat16)
                    d = rs_hop(r, s)
                    d.start()
                    rs[r].append(d)
                else:
                    ag_buf[r, 0, :, :] = merged.astype(jnp.bfloat16)
                    d = ag_hop(r, 0)
                    d.start()
                    ag[r].append(d)
                    out_ref[pl.ds(cidx(r, 3) * CHUNK, CHUNK),
                            r * COLS:(r + 1) * COLS] = merged.astype(jnp.bfloat16)

        for h in range(N_HOPS):
            for r in (0, 1):
                ag[r][h].wait_recv()
                if h < N_HOPS - 1:
                    d = ag_hop(r, h + 1)
                    d.start()
                    ag[r].append(d)
                out_ref[pl.ds(cidx(r, h) * CHUNK, CHUNK),
                        r * COLS:(r + 1) * COLS] = ag_buf[r, h + 1]

        for r in (0, 1):
            for d in rs[r] + ag[r]:
                d.wait_send()

    return pl.pallas_call(
        body,
        out_shape=jax.ShapeDtypeStruct((N_TOK, D_OUT), jnp.bfloat16),
        in_specs=[pl.BlockSpec(memory_space=pltpu.VMEM)] * 4,
        out_specs=pl.BlockSpec(memory_space=pltpu.VMEM),
        scratch_shapes=[
            pltpu.VMEM((N_TOK, E_LOC), jnp.float32),
            pltpu.VMEM((2, N_HOPS, CHUNK, COLS), jnp.bfloat16),
            pltpu.VMEM((2, N_HOPS, CHUNK, COLS), jnp.bfloat16),
            pltpu.VMEM((2, N_DEV, CHUNK, COLS), jnp.bfloat16),
            pltpu.SemaphoreType.DMA((2, N_HOPS)),
            pltpu.SemaphoreType.DMA((2, N_HOPS)),
            pltpu.SemaphoreType.DMA((2, N_HOPS)),
            pltpu.SemaphoreType.DMA((2, N_HOPS)),
        ],
        compiler_params=pltpu.CompilerParams(collective_id=0),
    )(x, router_W, route_idx, expert_W)


# device time: 39293 ns/iter; 1.0891x vs baseline; 1.0435x over previous
import jax
import jax.numpy as jnp
from jax import lax
from jax.experimental import pallas as pl
from jax.experimental.pallas import tpu as pltpu

N_DEV = 4
N_TOK = 1024
D_IN = 512
D_OUT = 1024
N_EXP = 16
E_LOC = N_EXP // N_DEV
N_HOPS = N_DEV - 1
CHUNK = N_TOK // N_DEV
COLS = D_OUT // 2


def kernel(x, router_W, route_idx, expert_W):
    def body(x_ref, rw_ref, idx_ref, ew_ref, out_ref,
             w_ref, rs_send, rs_recv, ag_own, ag_w1, ag_w2,
             rs_ssem, rs_rsem, ag_w1_ssem, ag_w1_rsem, ag_w2_ssem, ag_w2_rsem):
        my = lax.axis_index("i")
        left = lax.rem(my + N_DEV - 1, N_DEV)
        right = lax.rem(my + 1, N_DEV)
        peer = (right, left)

        barrier = pltpu.get_barrier_semaphore()
        for nbr in (left, right):
            pl.semaphore_signal(
                barrier, inc=1, device_id=(nbr,),
                device_id_type=pl.DeviceIdType.MESH,
            )
        pl.semaphore_wait(barrier, 2)

        def fill_gates(off, n):
            xc = x_ref[pl.ds(off, n), :]
            scores = jnp.dot(xc, rw_ref[:, :], preferred_element_type=jnp.float32)
            p = jnp.exp(scores - jnp.max(scores, axis=-1, keepdims=True))
            p = p / jnp.sum(p, axis=-1, keepdims=True)
            idx = idx_ref[pl.ds(off, n), :]
            eids = lax.broadcasted_iota(jnp.int32, (n, N_EXP), 1)
            g0 = jnp.sum(jnp.where(eids == idx[:, 0:1], p, 0.0),
                         axis=-1, keepdims=True)
            g1 = jnp.sum(jnp.where(eids == idx[:, 1:2], p, 0.0),
                         axis=-1, keepdims=True)
            inv = 1.0 / (g0 + g1)
            for k in range(E_LOC):
                e_k = my * E_LOC + k
                w_ref[pl.ds(off, n), k:k + 1] = (
                    jnp.where(idx[:, 0:1] == e_k, g0 * inv, 0.0)
                    + jnp.where(idx[:, 1:2] == e_k, g1 * inv, 0.0)
                )

        def gated_chunk(c):
            xc = x_ref[pl.ds(c * CHUNK, CHUNK), :]
            wc = w_ref[pl.ds(c * CHUNK, CHUNK), :]
            return jnp.concatenate(
                [(wc[:, k:k + 1] * xc).astype(jnp.bfloat16) for k in range(E_LOC)],
                axis=1,
            )

        def cidx(r, s):
            return lax.rem(my + (N_DEV - s if r == 0 else s), N_DEV)

        def rs_hop(r, s):
            return pltpu.make_async_remote_copy(
                src_ref=rs_send.at[r, s], dst_ref=rs_recv.at[r, s],
                send_sem=rs_ssem.at[r, s], recv_sem=rs_rsem.at[r, s],
                device_id=(peer[r],), device_id_type=pl.DeviceIdType.MESH,
            )

        def ag_wave1(r, to_right):
            side = 0 if to_right else 1
            return pltpu.make_async_remote_copy(
                src_ref=ag_own.at[r], dst_ref=ag_w1.at[r, side],
                send_sem=ag_w1_ssem.at[r, side], recv_sem=ag_w1_rsem.at[r, side],
                device_id=(right if to_right else left,),
                device_id_type=pl.DeviceIdType.MESH,
            )

        def ag_wave2(r):
            side = 0 if r == 0 else 1
            return pltpu.make_async_remote_copy(
                src_ref=ag_w1.at[r, side], dst_ref=ag_w2.at[r],
                send_sem=ag_w2_ssem.at[r], recv_sem=ag_w2_rsem.at[r],
                device_id=(peer[r],), device_id_type=pl.DeviceIdType.MESH,
            )

        fill_gates(my * CHUNK, CHUNK)
        ew_full = jnp.reshape(ew_ref[:, :, :], (E_LOC * D_IN, D_OUT))
        ew = [ew_full[:, r * COLS:(r + 1) * COLS].astype(jnp.bfloat16)
              for r in (0, 1)]
        gm = gated_chunk(my)
        rs = {0: [], 1: []}
        ag = {0: [], 1: []}
        for r in (0, 1):
            rs_send[r, 0, :, :] = jnp.dot(
                gm, ew[r], preferred_element_type=jnp.float32
            ).astype(jnp.bfloat16)
            d = rs_hop(r, 0)
            d.start()
            rs[r].append(d)

        fill_gates(lax.rem(my + 1, N_DEV) * CHUNK, CHUNK)
        fill_gates(lax.rem(my + 2, N_DEV) * CHUNK, CHUNK)
        fill_gates(lax.rem(my + 3, N_DEV) * CHUNK, CHUNK)

        for s in range(1, N_DEV):
            acc = {r: jnp.dot(gated_chunk(cidx(r, s)), ew[r],
                              preferred_element_type=jnp.float32)
                   for r in (0, 1)}
            for r in (0, 1):
                rs[r][s - 1].wait_recv()
                merged = rs_recv[r, s - 1].astype(jnp.float32) + acc[r]
                if s < N_DEV - 1:
                    rs_send[r, s, :, :] = merged.astype(jnp.bfloat16)
                    d = rs_hop(r, s)
                    d.start()
                    rs[r].append(d)
                else:
                    ag_own[r, :, :] = merged.astype(jnp.bfloat16)
                    dr = ag_wave1(r, to_right=True)
                    dl = ag_wave1(r, to_right=False)
                    dr.start()
                    dl.start()
                    ag[r] = [dr, dl]
                    out_ref[pl.ds(cidx(r, 3) * CHUNK, CHUNK),
                            r * COLS:(r + 1) * COLS] = merged.astype(jnp.bfloat16)

        def chunk_rows(delta):
            return pl.ds(lax.rem(my + delta, N_DEV) * CHUNK, CHUNK)

        w2 = []
        for r in (0, 1):
            side = 0 if r == 0 else 1
            ag[r][side].wait_recv()
            d = ag_wave2(r)
            d.start()
            w2.append(d)
        for r in (0, 1):
            fwd_side = 0 if r == 0 else 1
            ag[r][1 - fwd_side].wait_recv()
            out_ref[chunk_rows(0 if r == 0 else 2),
                    r * COLS:(r + 1) * COLS] = ag_w1[r, 0]
            out_ref[chunk_rows(2 if r == 0 else 0),
                    r * COLS:(r + 1) * COLS] = ag_w1[r, 1]
        for r in (0, 1):
            w2[r].wait_recv()
            out_ref[chunk_rows(3 if r == 0 else 1),
                    r * COLS:(r + 1) * COLS] = ag_w2[r]

        for r in (0, 1):
            for d in rs[r] + ag[r] + [w2[r]]:
                d.wait_send()

    return pl.pallas_call(
        body,
        out_shape=jax.ShapeDtypeStruct((N_TOK, D_OUT), jnp.bfloat16),
        in_specs=[pl.BlockSpec(memory_space=pltpu.VMEM)] * 4,
        out_specs=pl.BlockSpec(memory_space=pltpu.VMEM),
        scratch_shapes=[
            pltpu.VMEM((N_TOK, E_LOC), jnp.float32),
            pltpu.VMEM((2, N_HOPS, CHUNK, COLS), jnp.bfloat16),
            pltpu.VMEM((2, N_HOPS, CHUNK, COLS), jnp.bfloat16),
            pltpu.VMEM((2, CHUNK, COLS), jnp.bfloat16),
            pltpu.VMEM((2, 2, CHUNK, COLS), jnp.bfloat16),
            pltpu.VMEM((2, CHUNK, COLS), jnp.bfloat16),
            pltpu.SemaphoreType.DMA((2, N_HOPS)),
            pltpu.SemaphoreType.DMA((2, N_HOPS)),
            pltpu.SemaphoreType.DMA((2, 2)),
            pltpu.SemaphoreType.DMA((2, 2)),
            pltpu.SemaphoreType.DMA((2,)),
            pltpu.SemaphoreType.DMA((2,)),
        ],
        compiler_params=pltpu.CompilerParams(collective_id=0),
    )(x, router_W, route_idx, expert_W)
